# baseline (speedup 1.0000x reference)
import functools

import jax
import jax.numpy as jnp
from jax.experimental import pallas as pl
from jax.experimental.pallas import tpu as pltpu

_BIG = 3.0e38
_EPS_BN = 1e-5


def _pass_a(x1_ref, x2t_ref, p1_ref, p2_ref, w1at_ref, w1bt_ref,
            h1_ref, st_ref, p2w_ref, x2a_ref):
    b = pl.program_id(0)
    j = pl.program_id(1)

    @pl.when((b == 0) & (j == 0))
    def _init_stats():
        st_ref[...] = jnp.zeros_like(st_ref)

    @pl.when(j == 0)
    def _build_table():
        p2w_ref[...] = jnp.dot(p2_ref[0], w1bt_ref[...],
                               preferred_element_type=jnp.float32)
        x2t = x2t_ref[0]
        n2 = jnp.sum(x2t * x2t, axis=0, keepdims=True)
        x2a_ref[...] = jnp.concatenate(
            [-2.0 * x2t[0:3], n2, jnp.zeros((4, n2.shape[1]), jnp.float32)],
            axis=0)

    x1 = x1_ref[0]
    n1 = jnp.sum(x1 * x1, axis=1, keepdims=True) - 1.0
    e = jnp.dot(x1, x2a_ref[...], preferred_element_type=jnp.float32,
                precision=jax.lax.Precision.HIGHEST)

    r1 = e[:, 0:128]
    r2 = jnp.full_like(r1, _BIG)
    r3 = jnp.full_like(r1, _BIG)
    for k in range(1, e.shape[1] // 128):
        v = e[:, k * 128:(k + 1) * 128]
        lo = jnp.minimum(r1, v)
        hi = jnp.maximum(r1, v)
        lo2 = jnp.minimum(r2, hi)
        hi2 = jnp.maximum(r2, hi)
        r1, r2, r3 = lo, lo2, jnp.minimum(r3, hi2)
    cand = jnp.concatenate([r1, r2, r3], axis=1)
    c1 = jnp.min(cand, axis=1, keepdims=True)
    ca = jnp.where(cand == c1, _BIG, cand)
    c2 = jnp.min(ca, axis=1, keepdims=True)
    cb = jnp.where(ca == c2, _BIG, ca)
    m3 = jnp.min(cb, axis=1, keepdims=True)

    w = jax.lax.rsqrt(jnp.maximum(e + n1, 1e-12))
    wraw = jnp.where(e <= m3, w, 0.0)
    rs = jnp.sum(wraw, axis=1, keepdims=True)

    interp_h = jnp.dot(wraw, p2w_ref[...],
                       preferred_element_type=jnp.float32) / rs
    h1 = jnp.dot(p1_ref[0], w1at_ref[...],
                 preferred_element_type=jnp.float32) + interp_h
    h1_ref[0] = h1

    s = jnp.sum(h1, axis=0, keepdims=True)
    sq = jnp.sum(h1 * h1, axis=0, keepdims=True)
    st_ref[...] += jnp.concatenate(
        [s, sq, jnp.zeros((6, s.shape[1]), jnp.float32)], axis=0)


def _pass_b(h1_ref, st1_ref, g1_ref, b1_ref, w2t_ref, h2_ref, st2_ref,
            *, cnt):
    b = pl.program_id(0)
    j = pl.program_id(1)

    @pl.when((b == 0) & (j == 0))
    def _init_stats():
        st2_ref[...] = jnp.zeros_like(st2_ref)

    st = st1_ref[...]
    m = st[0:1, :] / cnt
    v = st[1:2, :] / cnt - m * m
    a = g1_ref[...] * jax.lax.rsqrt(v + _EPS_BN)
    c = b1_ref[...] - m * a
    y = jnp.maximum(h1_ref[0] * a + c, 0.0)
    h2 = jnp.dot(y, w2t_ref[...], preferred_element_type=jnp.float32)
    h2_ref[0] = h2

    s = jnp.sum(h2, axis=0, keepdims=True)
    sq = jnp.sum(h2 * h2, axis=0, keepdims=True)
    st2_ref[...] += jnp.concatenate(
        [s, sq, jnp.zeros((6, s.shape[1]), jnp.float32)], axis=0)


def _pass_c(h2_ref, st2_ref, g2_ref, b2_ref, out_ref, *, cnt):
    st = st2_ref[...]
    m = st[0:1, :] / cnt
    v = st[1:2, :] / cnt - m * m
    a = g2_ref[...] * jax.lax.rsqrt(v + _EPS_BN)
    c = b2_ref[...] - m * a
    out_ref[0] = jnp.maximum(h2_ref[0] * a + c, 0.0)


def kernel(xyz1, xyz2, points1, points2, W1, g1, b1, W2, g2, b2):
    B, N, _ = xyz1.shape
    S = xyz2.shape[1]
    C1 = points1.shape[-1]
    CO = W1.shape[0]
    C2 = points2.shape[-1]
    BN = 512
    NJ = N // BN
    cnt = float(B * N)

    ones = jnp.ones((B, N, 1), jnp.float32)
    zeros = jnp.zeros((B, N, 4), jnp.float32)
    xyz1p = jnp.concatenate([xyz1, ones, zeros], axis=-1)
    xyz2t = jnp.pad(xyz2, ((0, 0), (0, 0), (0, 5))).transpose(0, 2, 1)
    w1at = W1[:, :C1].T
    w1bt = W1[:, C1:].T
    w2t = W2.T
    g1r, b1r = g1.reshape(1, CO), b1.reshape(1, CO)
    g2r, b2r = g2.reshape(1, CO), b2.reshape(1, CO)

    h1, st1 = pl.pallas_call(
        _pass_a,
        grid=(B, NJ),
        in_specs=[
            pl.BlockSpec((1, BN, 8), lambda b, j: (b, j, 0)),
            pl.BlockSpec((1, 8, S), lambda b, j: (b, 0, 0)),
            pl.BlockSpec((1, BN, C1), lambda b, j: (b, j, 0)),
            pl.BlockSpec((1, S, C2), lambda b, j: (b, 0, 0)),
            pl.BlockSpec((C1, CO), lambda b, j: (0, 0)),
            pl.BlockSpec((C2, CO), lambda b, j: (0, 0)),
        ],
        out_specs=[
            pl.BlockSpec((1, BN, CO), lambda b, j: (b, j, 0)),
            pl.BlockSpec((8, CO), lambda b, j: (0, 0)),
        ],
        out_shape=[
            jax.ShapeDtypeStruct((B, N, CO), jnp.float32),
            jax.ShapeDtypeStruct((8, CO), jnp.float32),
        ],
        scratch_shapes=[pltpu.VMEM((S, CO), jnp.float32),
                        pltpu.VMEM((8, S), jnp.float32)],
    )(xyz1p, xyz2t, points1, points2, w1at, w1bt)

    h2, st2 = pl.pallas_call(
        functools.partial(_pass_b, cnt=cnt),
        grid=(B, NJ),
        in_specs=[
            pl.BlockSpec((1, BN, CO), lambda b, j: (b, j, 0)),
            pl.BlockSpec((8, CO), lambda b, j: (0, 0)),
            pl.BlockSpec((1, CO), lambda b, j: (0, 0)),
            pl.BlockSpec((1, CO), lambda b, j: (0, 0)),
            pl.BlockSpec((CO, CO), lambda b, j: (0, 0)),
        ],
        out_specs=[
            pl.BlockSpec((1, BN, CO), lambda b, j: (b, j, 0)),
            pl.BlockSpec((8, CO), lambda b, j: (0, 0)),
        ],
        out_shape=[
            jax.ShapeDtypeStruct((B, N, CO), jnp.float32),
            jax.ShapeDtypeStruct((8, CO), jnp.float32),
        ],
    )(h1, st1, g1r, b1r, w2t)

    out = pl.pallas_call(
        functools.partial(_pass_c, cnt=cnt),
        grid=(B, NJ),
        in_specs=[
            pl.BlockSpec((1, BN, CO), lambda b, j: (b, j, 0)),
            pl.BlockSpec((8, CO), lambda b, j: (0, 0)),
            pl.BlockSpec((1, CO), lambda b, j: (0, 0)),
            pl.BlockSpec((1, CO), lambda b, j: (0, 0)),
        ],
        out_specs=pl.BlockSpec((1, BN, CO), lambda b, j: (b, j, 0)),
        out_shape=jax.ShapeDtypeStruct((B, N, CO), jnp.float32),
    )(h2, st2, g2r, b2r)

    return out


# speedup vs baseline: 1.5009x; 1.5009x over previous
import functools

import jax
import jax.numpy as jnp
from jax.experimental import pallas as pl
from jax.experimental.pallas import tpu as pltpu

_BIG = 3.0e38
_EPS_BN = 1e-5


def _pass_a(x1_ref, x2t_ref, p1_ref, p2_ref, w1at_ref, w1bt_ref,
            h1_ref, st_ref, p2w_ref):
    b = pl.program_id(0)
    j = pl.program_id(1)

    @pl.when((b == 0) & (j == 0))
    def _init_stats():
        st_ref[...] = jnp.zeros_like(st_ref)

    @pl.when(j == 0)
    def _build_table():
        p2w_ref[...] = jnp.dot(p2_ref[0], w1bt_ref[...],
                               preferred_element_type=jnp.float32)

    x1 = x1_ref[0]
    x2t = x2t_ref[0]
    n1 = jnp.sum(x1 * x1, axis=1, keepdims=True)
    n2 = jnp.sum(x2t * x2t, axis=0, keepdims=True)
    g = jnp.dot(x1, x2t, preferred_element_type=jnp.float32)
    e = (n1 - 2.0 * g) + n2

    r1 = e[:, 0:128]
    r2 = jnp.full_like(r1, _BIG)
    r3 = jnp.full_like(r1, _BIG)
    for k in range(1, e.shape[1] // 128):
        v = e[:, k * 128:(k + 1) * 128]
        lo = jnp.minimum(r1, v)
        hi = jnp.maximum(r1, v)
        lo2 = jnp.minimum(r2, hi)
        hi2 = jnp.maximum(r2, hi)
        r1, r2, r3 = lo, lo2, jnp.minimum(r3, hi2)
    cand = jnp.concatenate([r1, r2, r3], axis=1)
    c1 = jnp.min(cand, axis=1, keepdims=True)
    ca = jnp.where(cand == c1, _BIG, cand)
    c2 = jnp.min(ca, axis=1, keepdims=True)
    cb = jnp.where(ca == c2, _BIG, ca)
    m3 = jnp.min(cb, axis=1, keepdims=True)

    w = jax.lax.rsqrt(jnp.maximum(e, 1e-12))
    wraw = jnp.where(e <= m3, w, 0.0)
    rs = jnp.sum(wraw, axis=1, keepdims=True)

    interp_h = jnp.dot(wraw, p2w_ref[...],
                       preferred_element_type=jnp.float32) / rs
    h1 = jnp.dot(p1_ref[0], w1at_ref[...],
                 preferred_element_type=jnp.float32) + interp_h
    h1_ref[0] = h1

    s = jnp.sum(h1, axis=0, keepdims=True)
    sq = jnp.sum(h1 * h1, axis=0, keepdims=True)
    st_ref[...] += jnp.concatenate(
        [s, sq, jnp.zeros((6, s.shape[1]), jnp.float32)], axis=0)


def _pass_b(h1_ref, st1_ref, g1_ref, b1_ref, w2t_ref, h2_ref, st2_ref,
            *, cnt):
    b = pl.program_id(0)
    j = pl.program_id(1)

    @pl.when((b == 0) & (j == 0))
    def _init_stats():
        st2_ref[...] = jnp.zeros_like(st2_ref)

    st = st1_ref[...]
    m = st[0:1, :] / cnt
    v = st[1:2, :] / cnt - m * m
    a = g1_ref[...] * jax.lax.rsqrt(v + _EPS_BN)
    c = b1_ref[...] - m * a
    y = jnp.maximum(h1_ref[0] * a + c, 0.0)
    h2 = jnp.dot(y, w2t_ref[...], preferred_element_type=jnp.float32)
    h2_ref[0] = h2

    s = jnp.sum(h2, axis=0, keepdims=True)
    sq = jnp.sum(h2 * h2, axis=0, keepdims=True)
    st2_ref[...] += jnp.concatenate(
        [s, sq, jnp.zeros((6, s.shape[1]), jnp.float32)], axis=0)


def _pass_c(h2_ref, st2_ref, g2_ref, b2_ref, out_ref, *, cnt):
    st = st2_ref[...]
    m = st[0:1, :] / cnt
    v = st[1:2, :] / cnt - m * m
    a = g2_ref[...] * jax.lax.rsqrt(v + _EPS_BN)
    c = b2_ref[...] - m * a
    out_ref[0] = jnp.maximum(h2_ref[0] * a + c, 0.0)


def kernel(xyz1, xyz2, points1, points2, W1, g1, b1, W2, g2, b2):
    B, N, _ = xyz1.shape
    S = xyz2.shape[1]
    C1 = points1.shape[-1]
    CO = W1.shape[0]
    C2 = points2.shape[-1]
    BN = 512
    NJ = N // BN
    cnt = float(B * N)

    xyz1p = jnp.pad(xyz1, ((0, 0), (0, 0), (0, 5)))
    xyz2t = jnp.pad(xyz2, ((0, 0), (0, 0), (0, 5))).transpose(0, 2, 1)
    w1at = W1[:, :C1].T
    w1bt = W1[:, C1:].T
    w2t = W2.T
    g1r, b1r = g1.reshape(1, CO), b1.reshape(1, CO)
    g2r, b2r = g2.reshape(1, CO), b2.reshape(1, CO)

    h1, st1 = pl.pallas_call(
        _pass_a,
        grid=(B, NJ),
        in_specs=[
            pl.BlockSpec((1, BN, 8), lambda b, j: (b, j, 0)),
            pl.BlockSpec((1, 8, S), lambda b, j: (b, 0, 0)),
            pl.BlockSpec((1, BN, C1), lambda b, j: (b, j, 0)),
            pl.BlockSpec((1, S, C2), lambda b, j: (b, 0, 0)),
            pl.BlockSpec((C1, CO), lambda b, j: (0, 0)),
            pl.BlockSpec((C2, CO), lambda b, j: (0, 0)),
        ],
        out_specs=[
            pl.BlockSpec((1, BN, CO), lambda b, j: (b, j, 0)),
            pl.BlockSpec((8, CO), lambda b, j: (0, 0)),
        ],
        out_shape=[
            jax.ShapeDtypeStruct((B, N, CO), jnp.float32),
            jax.ShapeDtypeStruct((8, CO), jnp.float32),
        ],
        scratch_shapes=[pltpu.VMEM((S, CO), jnp.float32)],
    )(xyz1p, xyz2t, points1, points2, w1at, w1bt)

    h2, st2 = pl.pallas_call(
        functools.partial(_pass_b, cnt=cnt),
        grid=(B, NJ),
        in_specs=[
            pl.BlockSpec((1, BN, CO), lambda b, j: (b, j, 0)),
            pl.BlockSpec((8, CO), lambda b, j: (0, 0)),
            pl.BlockSpec((1, CO), lambda b, j: (0, 0)),
            pl.BlockSpec((1, CO), lambda b, j: (0, 0)),
            pl.BlockSpec((CO, CO), lambda b, j: (0, 0)),
        ],
        out_specs=[
            pl.BlockSpec((1, BN, CO), lambda b, j: (b, j, 0)),
            pl.BlockSpec((8, CO), lambda b, j: (0, 0)),
        ],
        out_shape=[
            jax.ShapeDtypeStruct((B, N, CO), jnp.float32),
            jax.ShapeDtypeStruct((8, CO), jnp.float32),
        ],
    )(h1, st1, g1r, b1r, w2t)

    out = pl.pallas_call(
        functools.partial(_pass_c, cnt=cnt),
        grid=(B, NJ),
        in_specs=[
            pl.BlockSpec((1, BN, CO), lambda b, j: (b, j, 0)),
            pl.BlockSpec((8, CO), lambda b, j: (0, 0)),
            pl.BlockSpec((1, CO), lambda b, j: (0, 0)),
            pl.BlockSpec((1, CO), lambda b, j: (0, 0)),
        ],
        out_specs=pl.BlockSpec((1, BN, CO), lambda b, j: (b, j, 0)),
        out_shape=jax.ShapeDtypeStruct((B, N, CO), jnp.float32),
    )(h2, st2, g2r, b2r)

    return out


# speedup vs baseline: 1.5893x; 1.0590x over previous
import functools

import jax
import jax.numpy as jnp
from jax.experimental import pallas as pl
from jax.experimental.pallas import tpu as pltpu

_BIG = 3.0e38
_EPS_BN = 1e-5


def _pass_a(x1_ref, x2t_ref, p1_ref, p2_ref, w1at_ref, w1bt_ref,
            h1_ref, st_ref, p2w_ref):
    b = pl.program_id(0)
    j = pl.program_id(1)

    @pl.when((b == 0) & (j == 0))
    def _init_stats():
        st_ref[...] = jnp.zeros_like(st_ref)

    @pl.when(j == 0)
    def _build_table():
        p2w_ref[...] = jnp.dot(p2_ref[0], w1bt_ref[...],
                               preferred_element_type=jnp.float32)

    x1 = x1_ref[0]
    x2t = x2t_ref[0]
    n1 = jnp.sum(x1 * x1, axis=1, keepdims=True)
    n2 = jnp.sum(x2t * x2t, axis=0, keepdims=True)
    g = jnp.dot(x1, x2t, preferred_element_type=jnp.float32)
    e = (n1 - 2.0 * g) + n2

    r1 = e[:, 0:128]
    r2 = jnp.full_like(r1, _BIG)
    r3 = jnp.full_like(r1, _BIG)
    for k in range(1, e.shape[1] // 128):
        v = e[:, k * 128:(k + 1) * 128]
        lo = jnp.minimum(r1, v)
        hi = jnp.maximum(r1, v)
        lo2 = jnp.minimum(r2, hi)
        hi2 = jnp.maximum(r2, hi)
        r1, r2, r3 = lo, lo2, jnp.minimum(r3, hi2)
    cand = jnp.concatenate([r1, r2, r3], axis=1)
    c1 = jnp.min(cand, axis=1, keepdims=True)
    ca = jnp.where(cand == c1, _BIG, cand)
    c2 = jnp.min(ca, axis=1, keepdims=True)
    cb = jnp.where(ca == c2, _BIG, ca)
    m3 = jnp.min(cb, axis=1, keepdims=True)

    w = jax.lax.rsqrt(jnp.maximum(e, 1e-12))
    wraw = jnp.where(e <= m3, w, 0.0)
    rs = jnp.sum(wraw, axis=1, keepdims=True)

    interp_h = jnp.dot(wraw, p2w_ref[...],
                       preferred_element_type=jnp.float32) / rs
    h1 = jnp.dot(p1_ref[0], w1at_ref[...],
                 preferred_element_type=jnp.float32) + interp_h
    h1_ref[0] = h1

    s = jnp.sum(h1, axis=0, keepdims=True)
    sq = jnp.sum(h1 * h1, axis=0, keepdims=True)
    st_ref[...] += jnp.concatenate(
        [s, sq, jnp.zeros((6, s.shape[1]), jnp.float32)], axis=0)


def _bn_coeffs(st, g, b, cnt):
    m = st[0:1, :] / cnt
    v = st[1:2, :] / cnt - m * m
    a = g * jax.lax.rsqrt(v + _EPS_BN)
    return a, b - m * a


def _pass_bc(h1_ref, st1_ref, g1_ref, b1_ref, w2t_ref, g2_ref, b2_ref,
             out_ref, h2_scr, st2_scr, *, cnt, bn, nj):
    p = pl.program_id(0)
    b = pl.program_id(1)
    j = pl.program_id(2)
    base = (b * nj + j) * bn

    @pl.when(p == 0)
    def _phase_conv2():
        @pl.when((b == 0) & (j == 0))
        def _init_stats():
            st2_scr[...] = jnp.zeros_like(st2_scr)

        a, c = _bn_coeffs(st1_ref[...], g1_ref[...], b1_ref[...], cnt)
        y = jnp.maximum(h1_ref[0] * a + c, 0.0)
        h2 = jnp.dot(y, w2t_ref[...], preferred_element_type=jnp.float32)
        h2_scr[pl.ds(base, bn), :] = h2
        s = jnp.sum(h2, axis=0, keepdims=True)
        sq = jnp.sum(h2 * h2, axis=0, keepdims=True)
        st2_scr[...] += jnp.concatenate(
            [s, sq, jnp.zeros((6, s.shape[1]), jnp.float32)], axis=0)

    @pl.when(p == 1)
    def _phase_bn2():
        a, c = _bn_coeffs(st2_scr[...], g2_ref[...], b2_ref[...], cnt)
        out_ref[0] = jnp.maximum(h2_scr[pl.ds(base, bn), :] * a + c, 0.0)


def kernel(xyz1, xyz2, points1, points2, W1, g1, b1, W2, g2, b2):
    B, N, _ = xyz1.shape
    S = xyz2.shape[1]
    C1 = points1.shape[-1]
    CO = W1.shape[0]
    C2 = points2.shape[-1]
    BN = 512
    NJ = N // BN
    cnt = float(B * N)

    xyz1p = jnp.pad(xyz1, ((0, 0), (0, 0), (0, 5)))
    xyz2t = jnp.pad(xyz2, ((0, 0), (0, 0), (0, 5))).transpose(0, 2, 1)
    w1at = W1[:, :C1].T
    w1bt = W1[:, C1:].T
    w2t = W2.T
    g1r, b1r = g1.reshape(1, CO), b1.reshape(1, CO)
    g2r, b2r = g2.reshape(1, CO), b2.reshape(1, CO)

    h1, st1 = pl.pallas_call(
        _pass_a,
        grid=(B, NJ),
        in_specs=[
            pl.BlockSpec((1, BN, 8), lambda b, j: (b, j, 0)),
            pl.BlockSpec((1, 8, S), lambda b, j: (b, 0, 0)),
            pl.BlockSpec((1, BN, C1), lambda b, j: (b, j, 0)),
            pl.BlockSpec((1, S, C2), lambda b, j: (b, 0, 0)),
            pl.BlockSpec((C1, CO), lambda b, j: (0, 0)),
            pl.BlockSpec((C2, CO), lambda b, j: (0, 0)),
        ],
        out_specs=[
            pl.BlockSpec((1, BN, CO), lambda b, j: (b, j, 0)),
            pl.BlockSpec((8, CO), lambda b, j: (0, 0)),
        ],
        out_shape=[
            jax.ShapeDtypeStruct((B, N, CO), jnp.float32),
            jax.ShapeDtypeStruct((8, CO), jnp.float32),
        ],
        scratch_shapes=[pltpu.VMEM((S, CO), jnp.float32)],
    )(xyz1p, xyz2t, points1, points2, w1at, w1bt)

    out = pl.pallas_call(
        functools.partial(_pass_bc, cnt=cnt, bn=BN, nj=NJ),
        grid=(2, B, NJ),
        in_specs=[
            pl.BlockSpec(
                (1, BN, CO),
                lambda p, b, j: (jnp.where(p == 0, b, 0),
                                 jnp.where(p == 0, j, 0), 0)),
            pl.BlockSpec((8, CO), lambda p, b, j: (0, 0)),
            pl.BlockSpec((1, CO), lambda p, b, j: (0, 0)),
            pl.BlockSpec((1, CO), lambda p, b, j: (0, 0)),
            pl.BlockSpec((CO, CO), lambda p, b, j: (0, 0)),
            pl.BlockSpec((1, CO), lambda p, b, j: (0, 0)),
            pl.BlockSpec((1, CO), lambda p, b, j: (0, 0)),
        ],
        out_specs=pl.BlockSpec(
            (1, BN, CO),
            lambda p, b, j: (jnp.where(p == 1, b, 0),
                             jnp.where(p == 1, j, 0), 0)),
        out_shape=jax.ShapeDtypeStruct((B, N, CO), jnp.float32),
        scratch_shapes=[pltpu.VMEM((B * N, CO), jnp.float32),
                        pltpu.VMEM((8, CO), jnp.float32)],
    )(h1, st1, g1r, b1r, w2t, g2r, b2r)

    return out


# speedup vs baseline: 1.6620x; 1.0457x over previous
import functools

import jax
import jax.numpy as jnp
from jax.experimental import pallas as pl
from jax.experimental.pallas import tpu as pltpu

_BIG = 3.0e38
_EPS_BN = 1e-5


def _bn_coeffs(st, g, b, cnt):
    m = st[0:1, :] / cnt
    v = st[1:2, :] / cnt - m * m
    a = g * jax.lax.rsqrt(v + _EPS_BN)
    return a, b - m * a


def _stat_rows(h):
    s = jnp.sum(h, axis=0, keepdims=True)
    sq = jnp.sum(h * h, axis=0, keepdims=True)
    return jnp.concatenate(
        [s, sq, jnp.zeros((6, s.shape[1]), jnp.float32)], axis=0)


def _fused(x1_ref, x2t_ref, p1_ref, p2_ref, w1at_ref, w1bt_ref, w2t_ref,
           g1_ref, b1_ref, g2_ref, b2_ref, out_ref,
           p2w_ref, h1_scr, st1_scr, h2_scr, st2_scr, *, cnt, bn, nj):
    p = pl.program_id(0)
    b = pl.program_id(1)
    j = pl.program_id(2)
    base = (b * nj + j) * bn

    @pl.when(p == 0)
    def _phase_knn_conv1():
        @pl.when((b == 0) & (j == 0))
        def _init_stats():
            st1_scr[...] = jnp.zeros_like(st1_scr)
            st2_scr[...] = jnp.zeros_like(st2_scr)

        @pl.when(j == 0)
        def _build_table():
            p2w_ref[...] = jnp.dot(p2_ref[0], w1bt_ref[...],
                                   preferred_element_type=jnp.float32)

        x1 = x1_ref[0]
        x2t = x2t_ref[0]
        n1 = jnp.sum(x1 * x1, axis=1, keepdims=True)
        n2 = jnp.sum(x2t * x2t, axis=0, keepdims=True)
        g = jnp.dot(x1, x2t, preferred_element_type=jnp.float32)
        e = (n1 - 2.0 * g) + n2

        r1 = e[:, 0:128]
        r2 = jnp.full_like(r1, _BIG)
        r3 = jnp.full_like(r1, _BIG)
        for k in range(1, e.shape[1] // 128):
            v = e[:, k * 128:(k + 1) * 128]
            lo = jnp.minimum(r1, v)
            hi = jnp.maximum(r1, v)
            lo2 = jnp.minimum(r2, hi)
            hi2 = jnp.maximum(r2, hi)
            r1, r2, r3 = lo, lo2, jnp.minimum(r3, hi2)
        cand = jnp.concatenate([r1, r2, r3], axis=1)
        c1 = jnp.min(cand, axis=1, keepdims=True)
        ca = jnp.where(cand == c1, _BIG, cand)
        c2 = jnp.min(ca, axis=1, keepdims=True)
        cb = jnp.where(ca == c2, _BIG, ca)
        m3 = jnp.min(cb, axis=1, keepdims=True)

        w = jax.lax.rsqrt(jnp.maximum(e, 1e-12))
        wraw = jnp.where(e <= m3, w, 0.0)
        rs = jnp.sum(wraw, axis=1, keepdims=True)

        interp_h = jnp.dot(wraw, p2w_ref[...],
                           preferred_element_type=jnp.float32) / rs
        h1 = jnp.dot(p1_ref[0], w1at_ref[...],
                     preferred_element_type=jnp.float32) + interp_h
        h1_scr[pl.ds(base, bn), :] = h1
        st1_scr[...] += _stat_rows(h1)

    @pl.when(p == 1)
    def _phase_conv2():
        a, c = _bn_coeffs(st1_scr[...], g1_ref[...], b1_ref[...], cnt)
        y = jnp.maximum(h1_scr[pl.ds(base, bn), :] * a + c, 0.0)
        h2 = jnp.dot(y, w2t_ref[...], preferred_element_type=jnp.float32)
        h2_scr[pl.ds(base, bn), :] = h2
        st2_scr[...] += _stat_rows(h2)

    @pl.when(p == 2)
    def _phase_bn2():
        a, c = _bn_coeffs(st2_scr[...], g2_ref[...], b2_ref[...], cnt)
        out_ref[0] = jnp.maximum(h2_scr[pl.ds(base, bn), :] * a + c, 0.0)


def kernel(xyz1, xyz2, points1, points2, W1, g1, b1, W2, g2, b2):
    B, N, _ = xyz1.shape
    S = xyz2.shape[1]
    C1 = points1.shape[-1]
    CO = W1.shape[0]
    C2 = points2.shape[-1]
    BN = 512
    NJ = N // BN
    cnt = float(B * N)

    xyz1p = jnp.pad(xyz1, ((0, 0), (0, 0), (0, 5)))
    xyz2t = jnp.pad(xyz2, ((0, 0), (0, 0), (0, 5))).transpose(0, 2, 1)
    w1at = W1[:, :C1].T
    w1bt = W1[:, C1:].T
    w2t = W2.T
    g1r, b1r = g1.reshape(1, CO), b1.reshape(1, CO)
    g2r, b2r = g2.reshape(1, CO), b2.reshape(1, CO)

    out = pl.pallas_call(
        functools.partial(_fused, cnt=cnt, bn=BN, nj=NJ),
        grid=(3, B, NJ),
        in_specs=[
            pl.BlockSpec(
                (1, BN, 8),
                lambda p, b, j: (jnp.where(p == 0, b, 0),
                                 jnp.where(p == 0, j, 0), 0)),
            pl.BlockSpec(
                (1, 8, S),
                lambda p, b, j: (jnp.where(p == 0, b, 0), 0, 0)),
            pl.BlockSpec(
                (1, BN, C1),
                lambda p, b, j: (jnp.where(p == 0, b, 0),
                                 jnp.where(p == 0, j, 0), 0)),
            pl.BlockSpec(
                (1, S, C2),
                lambda p, b, j: (jnp.where(p == 0, b, 0), 0, 0)),
            pl.BlockSpec((C1, CO), lambda p, b, j: (0, 0)),
            pl.BlockSpec((C2, CO), lambda p, b, j: (0, 0)),
            pl.BlockSpec((CO, CO), lambda p, b, j: (0, 0)),
            pl.BlockSpec((1, CO), lambda p, b, j: (0, 0)),
            pl.BlockSpec((1, CO), lambda p, b, j: (0, 0)),
            pl.BlockSpec((1, CO), lambda p, b, j: (0, 0)),
            pl.BlockSpec((1, CO), lambda p, b, j: (0, 0)),
        ],
        out_specs=pl.BlockSpec(
            (1, BN, CO),
            lambda p, b, j: (jnp.where(p == 2, b, 0),
                             jnp.where(p == 2, j, 0), 0)),
        out_shape=jax.ShapeDtypeStruct((B, N, CO), jnp.float32),
        scratch_shapes=[
            pltpu.VMEM((S, CO), jnp.float32),
            pltpu.VMEM((B * N, CO), jnp.float32),
            pltpu.VMEM((8, CO), jnp.float32),
            pltpu.VMEM((B * N, CO), jnp.float32),
            pltpu.VMEM((8, CO), jnp.float32),
        ],
    )(xyz1p, xyz2t, points1, points2, w1at, w1bt, w2t,
      g1r, b1r, g2r, b2r)

    return out


# speedup vs baseline: 1.8479x; 1.1119x over previous
import functools

import jax
import jax.numpy as jnp
from jax.experimental import pallas as pl
from jax.experimental.pallas import tpu as pltpu

_BIG = 3.0e38
_EPS_BN = 1e-5


def _bn_coeffs(st, g, b, cnt):
    m = st[0:1, :] / cnt
    v = st[1:2, :] / cnt - m * m
    a = g * jax.lax.rsqrt(v + _EPS_BN)
    return a, b - m * a


def _stat_rows(h):
    s = jnp.sum(h, axis=0, keepdims=True)
    sq = jnp.sum(h * h, axis=0, keepdims=True)
    return jnp.concatenate(
        [s, sq, jnp.zeros((6, s.shape[1]), jnp.float32)], axis=0)


def _fused(x1_ref, x2t_ref, p1_ref, p2_ref, w1at_ref, w1bt_ref, w2t_ref,
           g1_ref, b1_ref, g2_ref, b2_ref, out_ref,
           p2w_ref, h1_scr, st1_scr, h2_scr, st2_scr, *, cnt, bn, nj):
    p = pl.program_id(0)
    b = pl.program_id(1)
    j = pl.program_id(2)
    base = (b * nj + j) * bn

    @pl.when(p == 0)
    def _phase_knn_conv1():
        @pl.when((b == 0) & (j == 0))
        def _init_stats():
            st1_scr[...] = jnp.zeros_like(st1_scr)
            st2_scr[...] = jnp.zeros_like(st2_scr)

        @pl.when(j == 0)
        def _build_table():
            p2w_ref[...] = jnp.dot(p2_ref[0], w1bt_ref[...],
                                   preferred_element_type=jnp.float32)

        x1 = x1_ref[0]
        x2t = x2t_ref[0]
        n1 = jnp.sum(x1 * x1, axis=1, keepdims=True)
        n2 = jnp.sum(x2t * x2t, axis=0, keepdims=True)
        g = jnp.dot(x1, x2t, preferred_element_type=jnp.float32)
        e = (n1 - 2.0 * g) + n2

        r1 = e[:, 0:128]
        r2 = jnp.full_like(r1, _BIG)
        r3 = jnp.full_like(r1, _BIG)
        for k in range(1, e.shape[1] // 128):
            v = e[:, k * 128:(k + 1) * 128]
            lo = jnp.minimum(r1, v)
            hi = jnp.maximum(r1, v)
            lo2 = jnp.minimum(r2, hi)
            hi2 = jnp.maximum(r2, hi)
            r1, r2, r3 = lo, lo2, jnp.minimum(r3, hi2)
        cand = jnp.concatenate([r1, r2, r3], axis=1)
        c1 = jnp.min(cand, axis=1, keepdims=True)
        ca = jnp.where(cand == c1, _BIG, cand)
        c2 = jnp.min(ca, axis=1, keepdims=True)
        cb = jnp.where(ca == c2, _BIG, ca)
        m3 = jnp.min(cb, axis=1, keepdims=True)

        w = jax.lax.rsqrt(jnp.maximum(e, 1e-12))
        wraw = jnp.where(e <= m3, w, 0.0)
        rs = jnp.sum(wraw, axis=1, keepdims=True)

        interp_h = jnp.dot(wraw, p2w_ref[...],
                           preferred_element_type=jnp.float32) / rs
        h1 = jnp.dot(p1_ref[0], w1at_ref[...],
                     preferred_element_type=jnp.float32) + interp_h
        h1_scr[pl.ds(base, bn), :] = h1
        st1_scr[...] += _stat_rows(h1)

    @pl.when(p == 1)
    def _phase_conv2():
        a, c = _bn_coeffs(st1_scr[...], g1_ref[...], b1_ref[...], cnt)
        y = jnp.maximum(h1_scr[pl.ds(base, bn), :] * a + c, 0.0)
        h2 = jnp.dot(y, w2t_ref[...], preferred_element_type=jnp.float32)
        h2_scr[pl.ds(base, bn), :] = h2
        st2_scr[...] += _stat_rows(h2)

    @pl.when(p == 2)
    def _phase_bn2():
        a, c = _bn_coeffs(st2_scr[...], g2_ref[...], b2_ref[...], cnt)
        out_ref[0] = jnp.maximum(h2_scr[pl.ds(base, bn), :] * a + c, 0.0)


def kernel(xyz1, xyz2, points1, points2, W1, g1, b1, W2, g2, b2):
    B, N, _ = xyz1.shape
    S = xyz2.shape[1]
    C1 = points1.shape[-1]
    CO = W1.shape[0]
    C2 = points2.shape[-1]
    BN = 1024
    NJ = N // BN
    cnt = float(B * N)

    xyz1p = jnp.pad(xyz1, ((0, 0), (0, 0), (0, 5)))
    xyz2t = jnp.pad(xyz2, ((0, 0), (0, 0), (0, 5))).transpose(0, 2, 1)
    w1at = W1[:, :C1].T
    w1bt = W1[:, C1:].T
    w2t = W2.T
    g1r, b1r = g1.reshape(1, CO), b1.reshape(1, CO)
    g2r, b2r = g2.reshape(1, CO), b2.reshape(1, CO)

    out = pl.pallas_call(
        functools.partial(_fused, cnt=cnt, bn=BN, nj=NJ),
        grid=(3, B, NJ),
        in_specs=[
            pl.BlockSpec(
                (1, BN, 8),
                lambda p, b, j: (jnp.where(p == 0, b, 0),
                                 jnp.where(p == 0, j, 0), 0)),
            pl.BlockSpec(
                (1, 8, S),
                lambda p, b, j: (jnp.where(p == 0, b, 0), 0, 0)),
            pl.BlockSpec(
                (1, BN, C1),
                lambda p, b, j: (jnp.where(p == 0, b, 0),
                                 jnp.where(p == 0, j, 0), 0)),
            pl.BlockSpec(
                (1, S, C2),
                lambda p, b, j: (jnp.where(p == 0, b, 0), 0, 0)),
            pl.BlockSpec((C1, CO), lambda p, b, j: (0, 0)),
            pl.BlockSpec((C2, CO), lambda p, b, j: (0, 0)),
            pl.BlockSpec((CO, CO), lambda p, b, j: (0, 0)),
            pl.BlockSpec((1, CO), lambda p, b, j: (0, 0)),
            pl.BlockSpec((1, CO), lambda p, b, j: (0, 0)),
            pl.BlockSpec((1, CO), lambda p, b, j: (0, 0)),
            pl.BlockSpec((1, CO), lambda p, b, j: (0, 0)),
        ],
        out_specs=pl.BlockSpec(
            (1, BN, CO),
            lambda p, b, j: (jnp.where(p == 2, b, 0),
                             jnp.where(p == 2, j, 0), 0)),
        out_shape=jax.ShapeDtypeStruct((B, N, CO), jnp.float32),
        scratch_shapes=[
            pltpu.VMEM((S, CO), jnp.float32),
            pltpu.VMEM((B * N, CO), jnp.float32),
            pltpu.VMEM((8, CO), jnp.float32),
            pltpu.VMEM((B * N, CO), jnp.float32),
            pltpu.VMEM((8, CO), jnp.float32),
        ],
    )(xyz1p, xyz2t, points1, points2, w1at, w1bt, w2t,
      g1r, b1r, g2r, b2r)

    return out
